# baseline (speedup 1.0000x reference)
import functools

import jax
import jax.numpy as jnp
from jax import lax
from jax.experimental import pallas as pl
from jax.experimental.pallas import tpu as pltpu
from jax.experimental.pallas import tpu_sc as plsc

_TOP_K = 2
_ROUTED_SCALING = 2.5
_TILE = 256


def _sc_row_gather(table_3d, idx, n_rows, chunk=32):
    info = plsc.get_sparse_core_info()
    nw = info.num_cores * info.num_subcores
    b_per_w = n_rows // nw
    n_chunks = b_per_w // chunk
    _, sl, lanes = table_3d.shape
    mesh = plsc.VectorSubcoreMesh(core_axis_name="c", subcore_axis_name="s")

    @functools.partial(
        pl.kernel, mesh=mesh,
        out_type=jax.ShapeDtypeStruct((n_rows, sl, lanes), table_3d.dtype),
        scratch_types=[
            pltpu.VMEM((chunk,), jnp.int32),
            pltpu.VMEM((chunk,), jnp.int32),
            pltpu.VMEM((chunk, sl, lanes), table_3d.dtype),
            pltpu.VMEM((chunk, sl, lanes), table_3d.dtype),
            pltpu.SemaphoreType.DMA,
            pltpu.SemaphoreType.DMA,
        ],
    )
    def gather_kernel(tbl_hbm, idx_hbm, out_hbm, i0, i1, b0, b1, s0, s1):
        wid = lax.axis_index("s") * info.num_cores + lax.axis_index("c")
        base = wid * b_per_w
        idx_v = (i0, i1)
        bufs = (b0, b1)
        sems = (s0, s1)

        def start(c):
            k = c % 2
            pltpu.sync_copy(idx_hbm.at[pl.ds(base + c * chunk, chunk)],
                            idx_v[k])
            return pltpu.async_copy(tbl_hbm.at[idx_v[k]], bufs[k], sems[k])

        cp = start(0)
        for c in range(n_chunks):
            nxt = start(c + 1) if c + 1 < n_chunks else None
            cp.wait()
            pltpu.sync_copy(bufs[c % 2],
                            out_hbm.at[pl.ds(base + c * chunk, chunk)])
            cp = nxt

    return gather_kernel(table_3d, idx)


def _moe_tile_kernel(te_ref, x_ref, wg_ref, wu_ref, wd_ref, o_ref):
    x = x_ref[...].astype(jnp.bfloat16)
    g = jnp.dot(x, wg_ref[0], preferred_element_type=jnp.float32)
    u = jnp.dot(x, wu_ref[0], preferred_element_type=jnp.float32)
    h = (g * jax.nn.sigmoid(g)) * u
    o_ref[...] = jnp.dot(h.astype(jnp.bfloat16), wd_ref[0],
                         preferred_element_type=jnp.float32)


def _shared_tile_kernel(x_ref, sg_ref, su_ref, sd_ref, o_ref):
    x = x_ref[...].astype(jnp.bfloat16)
    g = jnp.dot(x, sg_ref[...], preferred_element_type=jnp.float32)
    u = jnp.dot(x, su_ref[...], preferred_element_type=jnp.float32)
    h = (g * jax.nn.sigmoid(g)) * u
    o_ref[...] = jnp.dot(h.astype(jnp.bfloat16), sd_ref[...],
                         preferred_element_type=jnp.float32)


def kernel(hidden_states, gate_weight, gate_bias, all_gate_proj,
           all_up_proj, all_down_proj, shared_gate, shared_up, shared_down):
    orig_shape = hidden_states.shape
    D = orig_shape[-1]
    h = hidden_states.reshape(-1, D)
    T = h.shape[0]
    E, _, F = all_gate_proj.shape
    FS = shared_gate.shape[1]
    P = T * _TOP_K

    scores = jax.nn.sigmoid(h @ gate_weight)
    _, topk_idx = jax.lax.top_k(scores + gate_bias[None, :], _TOP_K)
    topk_w = jnp.take_along_axis(scores, topk_idx, axis=1)
    topk_w = topk_w / (jnp.sum(topk_w, axis=-1, keepdims=True) + 1e-20)
    topk_w = topk_w * _ROUTED_SCALING

    e_flat = topk_idx.reshape(-1).astype(jnp.int32)
    onehot_t = (e_flat[None, :] == jnp.arange(E, dtype=jnp.int32)[:, None]
                ).astype(jnp.int32)
    cum_t = jnp.cumsum(onehot_t, axis=1)
    rank = jnp.take_along_axis(cum_t, e_flat[None, :], axis=0)[0] - 1
    gsz = cum_t[:, -1]

    max_tiles = P // _TILE + E
    npad = max_tiles * _TILE

    nt = (gsz + _TILE - 1) // _TILE
    nt_cum = jnp.cumsum(nt)
    pad_base = (nt_cum - nt) * _TILE
    tile_expert = jnp.minimum(
        jnp.searchsorted(nt_cum, jnp.arange(max_tiles, dtype=jnp.int32),
                         side='right'),
        E - 1).astype(jnp.int32)

    dest = pad_base[e_flat] + rank
    slot_token = jnp.zeros((npad,), jnp.int32)
    tok_of_pair = jnp.arange(P, dtype=jnp.int32) // _TOP_K
    slot_token = slot_token.at[dest].set(tok_of_pair)

    x_pad = _sc_row_gather(h.reshape(T, D // 128, 128), slot_token,
                           npad).reshape(npad, D)
    wg = all_gate_proj.astype(jnp.bfloat16)
    wu = all_up_proj.astype(jnp.bfloat16)
    wd = all_down_proj.astype(jnp.bfloat16)

    grid_spec = pltpu.PrefetchScalarGridSpec(
        num_scalar_prefetch=1,
        grid=(max_tiles,),
        in_specs=[
            pl.BlockSpec((_TILE, D), lambda i, te: (i, 0)),
            pl.BlockSpec((1, D, F), lambda i, te: (te[i], 0, 0)),
            pl.BlockSpec((1, D, F), lambda i, te: (te[i], 0, 0)),
            pl.BlockSpec((1, F, D), lambda i, te: (te[i], 0, 0)),
        ],
        out_specs=pl.BlockSpec((_TILE, D), lambda i, te: (i, 0)),
    )
    y_pad = pl.pallas_call(
        _moe_tile_kernel,
        grid_spec=grid_spec,
        out_shape=jax.ShapeDtypeStruct((npad, D), jnp.float32),
    )(tile_expert, x_pad, wg, wu, wd)

    shared_out = pl.pallas_call(
        _shared_tile_kernel,
        grid=(T // _TILE,),
        in_specs=[
            pl.BlockSpec((_TILE, D), lambda i: (i, 0)),
            pl.BlockSpec((D, FS), lambda i: (0, 0)),
            pl.BlockSpec((D, FS), lambda i: (0, 0)),
            pl.BlockSpec((FS, D), lambda i: (0, 0)),
        ],
        out_specs=pl.BlockSpec((_TILE, D), lambda i: (i, 0)),
        out_shape=jax.ShapeDtypeStruct((T, D), jnp.float32),
    )(h, shared_gate.astype(jnp.bfloat16),
      shared_up.astype(jnp.bfloat16), shared_down.astype(jnp.bfloat16))

    dest_km = dest.reshape(T, _TOP_K).T.reshape(P)
    y_routed = _sc_row_gather(y_pad.reshape(npad, D // 128, 128), dest_km,
                              P).reshape(_TOP_K, T, D)
    w_km = topk_w.T
    out = shared_out + jnp.sum(w_km[:, :, None] * y_routed, axis=0)
    return out.reshape(orig_shape)


# speedup vs baseline: 1.4869x; 1.4869x over previous
import functools

import jax
import jax.numpy as jnp
from jax import lax
from jax.experimental import pallas as pl
from jax.experimental.pallas import tpu as pltpu
from jax.experimental.pallas import tpu_sc as plsc

_TOP_K = 2
_ROUTED_SCALING = 2.5
_TILE = 256


def _sc_row_gather(table_3d, idx, n_rows, chunk=32):
    info = plsc.get_sparse_core_info()
    nw = info.num_cores * info.num_subcores
    b_per_w = n_rows // nw
    n_chunks = b_per_w // chunk
    _, sl, lanes = table_3d.shape
    mesh = plsc.VectorSubcoreMesh(core_axis_name="c", subcore_axis_name="s")

    @functools.partial(
        pl.kernel, mesh=mesh,
        out_type=jax.ShapeDtypeStruct((n_rows, sl, lanes), table_3d.dtype),
        scratch_types=[
            pltpu.VMEM((chunk,), jnp.int32),
            pltpu.VMEM((chunk,), jnp.int32),
            pltpu.VMEM((chunk, sl, lanes), table_3d.dtype),
            pltpu.VMEM((chunk, sl, lanes), table_3d.dtype),
            pltpu.SemaphoreType.DMA,
            pltpu.SemaphoreType.DMA,
        ],
    )
    def gather_kernel(tbl_hbm, idx_hbm, out_hbm, i0, i1, b0, b1, s0, s1):
        wid = lax.axis_index("s") * info.num_cores + lax.axis_index("c")
        base = wid * b_per_w
        idx_v = (i0, i1)
        bufs = (b0, b1)
        sems = (s0, s1)

        def start(c):
            k = c % 2
            pltpu.sync_copy(idx_hbm.at[pl.ds(base + c * chunk, chunk)],
                            idx_v[k])
            return pltpu.async_copy(tbl_hbm.at[idx_v[k]], bufs[k], sems[k])

        cp = start(0)
        for c in range(n_chunks):
            nxt = start(c + 1) if c + 1 < n_chunks else None
            cp.wait()
            pltpu.sync_copy(bufs[c % 2],
                            out_hbm.at[pl.ds(base + c * chunk, chunk)])
            cp = nxt

    return gather_kernel(table_3d, idx)


def _moe_tile_kernel(te_ref, x_ref, wg_ref, wu_ref, wd_ref, o_ref):
    x = x_ref[...].astype(jnp.bfloat16)
    g = jnp.dot(x, wg_ref[0], preferred_element_type=jnp.float32)
    u = jnp.dot(x, wu_ref[0], preferred_element_type=jnp.float32)
    h = (g * jax.nn.sigmoid(g)) * u
    o_ref[...] = jnp.dot(h.astype(jnp.bfloat16), wd_ref[0],
                         preferred_element_type=jnp.float32)


def _shared_tile_kernel(x_ref, sg_ref, su_ref, sd_ref, o_ref):
    x = x_ref[...].astype(jnp.bfloat16)
    g = jnp.dot(x, sg_ref[...], preferred_element_type=jnp.float32)
    u = jnp.dot(x, su_ref[...], preferred_element_type=jnp.float32)
    h = (g * jax.nn.sigmoid(g)) * u
    o_ref[...] = jnp.dot(h.astype(jnp.bfloat16), sd_ref[...],
                         preferred_element_type=jnp.float32)


def kernel(hidden_states, gate_weight, gate_bias, all_gate_proj,
           all_up_proj, all_down_proj, shared_gate, shared_up, shared_down):
    orig_shape = hidden_states.shape
    D = orig_shape[-1]
    h = hidden_states.reshape(-1, D)
    T = h.shape[0]
    E, _, F = all_gate_proj.shape
    FS = shared_gate.shape[1]
    P = T * _TOP_K

    scores = jax.nn.sigmoid(h @ gate_weight)
    _, topk_idx = jax.lax.top_k(scores + gate_bias[None, :], _TOP_K)
    topk_w = jnp.take_along_axis(scores, topk_idx, axis=1)
    topk_w = topk_w / (jnp.sum(topk_w, axis=-1, keepdims=True) + 1e-20)
    topk_w = topk_w * _ROUTED_SCALING

    e_flat = topk_idx.reshape(-1).astype(jnp.int32)
    onehot_t = (e_flat[None, :] == jnp.arange(E, dtype=jnp.int32)[:, None]
                ).astype(jnp.int32)
    cum_t = jnp.cumsum(onehot_t, axis=1)
    rank = jnp.take_along_axis(cum_t, e_flat[None, :], axis=0)[0] - 1
    gsz = cum_t[:, -1]

    max_tiles = P // _TILE + E
    npad = max_tiles * _TILE

    nt = (gsz + _TILE - 1) // _TILE
    nt_cum = jnp.cumsum(nt)
    pad_base = (nt_cum - nt) * _TILE
    tile_expert = jnp.minimum(
        jnp.searchsorted(nt_cum, jnp.arange(max_tiles, dtype=jnp.int32),
                         side='right'),
        E - 1).astype(jnp.int32)

    dest = pad_base[e_flat] + rank
    slot_token = jnp.arange(npad, dtype=jnp.int32) % T
    tok_of_pair = jnp.arange(P, dtype=jnp.int32) // _TOP_K
    slot_token = slot_token.at[dest].set(tok_of_pair)

    x_pad = _sc_row_gather(h.reshape(T, D // 128, 128), slot_token,
                           npad).reshape(npad, D)
    wg = all_gate_proj.astype(jnp.bfloat16)
    wu = all_up_proj.astype(jnp.bfloat16)
    wd = all_down_proj.astype(jnp.bfloat16)

    grid_spec = pltpu.PrefetchScalarGridSpec(
        num_scalar_prefetch=1,
        grid=(max_tiles,),
        in_specs=[
            pl.BlockSpec((_TILE, D), lambda i, te: (i, 0)),
            pl.BlockSpec((1, D, F), lambda i, te: (te[i], 0, 0)),
            pl.BlockSpec((1, D, F), lambda i, te: (te[i], 0, 0)),
            pl.BlockSpec((1, F, D), lambda i, te: (te[i], 0, 0)),
        ],
        out_specs=pl.BlockSpec((_TILE, D), lambda i, te: (i, 0)),
    )
    y_pad = pl.pallas_call(
        _moe_tile_kernel,
        grid_spec=grid_spec,
        out_shape=jax.ShapeDtypeStruct((npad, D), jnp.float32),
    )(tile_expert, x_pad, wg, wu, wd)

    shared_out = pl.pallas_call(
        _shared_tile_kernel,
        grid=(T // _TILE,),
        in_specs=[
            pl.BlockSpec((_TILE, D), lambda i: (i, 0)),
            pl.BlockSpec((D, FS), lambda i: (0, 0)),
            pl.BlockSpec((D, FS), lambda i: (0, 0)),
            pl.BlockSpec((FS, D), lambda i: (0, 0)),
        ],
        out_specs=pl.BlockSpec((_TILE, D), lambda i: (i, 0)),
        out_shape=jax.ShapeDtypeStruct((T, D), jnp.float32),
    )(h, shared_gate.astype(jnp.bfloat16),
      shared_up.astype(jnp.bfloat16), shared_down.astype(jnp.bfloat16))

    dest_km = dest.reshape(T, _TOP_K).T.reshape(P)
    y_routed = _sc_row_gather(y_pad.reshape(npad, D // 128, 128), dest_km,
                              P).reshape(_TOP_K, T, D)
    w_km = topk_w.T
    out = shared_out + jnp.sum(w_km[:, :, None] * y_routed, axis=0)
    return out.reshape(orig_shape)


# speedup vs baseline: 1.7600x; 1.1837x over previous
import functools

import jax
import jax.numpy as jnp
from jax import lax
from jax.experimental import pallas as pl
from jax.experimental.pallas import tpu as pltpu
from jax.experimental.pallas import tpu_sc as plsc

_TOP_K = 2
_ROUTED_SCALING = 2.5
_TILE = 256


def _sc_row_gather(table, idx, n_rows, chunk=32):
    info = plsc.get_sparse_core_info()
    nw = info.num_cores * info.num_subcores
    b_per_w = n_rows // nw
    n_chunks = b_per_w // chunk
    _, d = table.shape
    mesh = plsc.VectorSubcoreMesh(core_axis_name="c", subcore_axis_name="s")

    @functools.partial(
        pl.kernel, mesh=mesh,
        out_type=jax.ShapeDtypeStruct((n_rows, d), table.dtype),
        scratch_types=[
            pltpu.VMEM((chunk,), jnp.int32),
            pltpu.VMEM((chunk,), jnp.int32),
            pltpu.VMEM((chunk, d), table.dtype),
            pltpu.VMEM((chunk, d), table.dtype),
            pltpu.SemaphoreType.DMA,
            pltpu.SemaphoreType.DMA,
        ],
    )
    def gather_kernel(tbl_hbm, idx_hbm, out_hbm, i0, i1, b0, b1, s0, s1):
        wid = lax.axis_index("s") * info.num_cores + lax.axis_index("c")
        base = wid * b_per_w
        idx_v = (i0, i1)
        bufs = (b0, b1)
        sems = (s0, s1)

        def start(c):
            k = c % 2
            pltpu.sync_copy(idx_hbm.at[pl.ds(base + c * chunk, chunk)],
                            idx_v[k])
            return pltpu.async_copy(tbl_hbm.at[idx_v[k]], bufs[k], sems[k])

        cp = start(0)
        for c in range(n_chunks):
            nxt = start(c + 1) if c + 1 < n_chunks else None
            cp.wait()
            pltpu.sync_copy(bufs[c % 2],
                            out_hbm.at[pl.ds(base + c * chunk, chunk)])
            cp = nxt

    return gather_kernel(table, idx)


def _moe_tile_kernel(te_ref, x_ref, wg_ref, wu_ref, wd_ref, o_ref):
    x = x_ref[...].astype(jnp.bfloat16)
    g = jnp.dot(x, wg_ref[0], preferred_element_type=jnp.float32)
    u = jnp.dot(x, wu_ref[0], preferred_element_type=jnp.float32)
    h = (g * jax.nn.sigmoid(g)) * u
    o_ref[...] = jnp.dot(h.astype(jnp.bfloat16), wd_ref[0],
                         preferred_element_type=jnp.float32)


def _shared_tile_kernel(x_ref, sg_ref, su_ref, sd_ref, o_ref):
    x = x_ref[...].astype(jnp.bfloat16)
    g = jnp.dot(x, sg_ref[...], preferred_element_type=jnp.float32)
    u = jnp.dot(x, su_ref[...], preferred_element_type=jnp.float32)
    h = (g * jax.nn.sigmoid(g)) * u
    o_ref[...] = jnp.dot(h.astype(jnp.bfloat16), sd_ref[...],
                         preferred_element_type=jnp.float32)


def kernel(hidden_states, gate_weight, gate_bias, all_gate_proj,
           all_up_proj, all_down_proj, shared_gate, shared_up, shared_down):
    orig_shape = hidden_states.shape
    D = orig_shape[-1]
    h = hidden_states.reshape(-1, D)
    T = h.shape[0]
    E, _, F = all_gate_proj.shape
    FS = shared_gate.shape[1]
    P = T * _TOP_K

    scores = jax.nn.sigmoid(h @ gate_weight)
    _, topk_idx = jax.lax.top_k(scores + gate_bias[None, :], _TOP_K)
    topk_w = jnp.take_along_axis(scores, topk_idx, axis=1)
    topk_w = topk_w / (jnp.sum(topk_w, axis=-1, keepdims=True) + 1e-20)
    topk_w = topk_w * _ROUTED_SCALING

    e_flat = topk_idx.reshape(-1).astype(jnp.int32)
    onehot_t = (e_flat[None, :] == jnp.arange(E, dtype=jnp.int32)[:, None]
                ).astype(jnp.int32)
    cum_t = jnp.cumsum(onehot_t, axis=1)
    rank = jnp.take_along_axis(cum_t, e_flat[None, :], axis=0)[0] - 1
    gsz = cum_t[:, -1]

    max_tiles = P // _TILE + E
    npad = max_tiles * _TILE

    nt = (gsz + _TILE - 1) // _TILE
    nt_cum = jnp.cumsum(nt)
    pad_base = (nt_cum - nt) * _TILE
    tile_expert = jnp.minimum(
        jnp.searchsorted(nt_cum, jnp.arange(max_tiles, dtype=jnp.int32),
                         side='right'),
        E - 1).astype(jnp.int32)

    dest = pad_base[e_flat] + rank
    slot_token = jnp.arange(npad, dtype=jnp.int32) % T
    tok_of_pair = jnp.arange(P, dtype=jnp.int32) // _TOP_K
    slot_token = slot_token.at[dest].set(tok_of_pair)

    x_pad = _sc_row_gather(h, slot_token, npad)
    wg = all_gate_proj.astype(jnp.bfloat16)
    wu = all_up_proj.astype(jnp.bfloat16)
    wd = all_down_proj.astype(jnp.bfloat16)

    grid_spec = pltpu.PrefetchScalarGridSpec(
        num_scalar_prefetch=1,
        grid=(max_tiles,),
        in_specs=[
            pl.BlockSpec((_TILE, D), lambda i, te: (i, 0)),
            pl.BlockSpec((1, D, F), lambda i, te: (te[i], 0, 0)),
            pl.BlockSpec((1, D, F), lambda i, te: (te[i], 0, 0)),
            pl.BlockSpec((1, F, D), lambda i, te: (te[i], 0, 0)),
        ],
        out_specs=pl.BlockSpec((_TILE, D), lambda i, te: (i, 0)),
    )
    y_pad = pl.pallas_call(
        _moe_tile_kernel,
        grid_spec=grid_spec,
        out_shape=jax.ShapeDtypeStruct((npad, D), jnp.float32),
    )(tile_expert, x_pad, wg, wu, wd)

    shared_out = pl.pallas_call(
        _shared_tile_kernel,
        grid=(T // _TILE,),
        in_specs=[
            pl.BlockSpec((_TILE, D), lambda i: (i, 0)),
            pl.BlockSpec((D, FS), lambda i: (0, 0)),
            pl.BlockSpec((D, FS), lambda i: (0, 0)),
            pl.BlockSpec((FS, D), lambda i: (0, 0)),
        ],
        out_specs=pl.BlockSpec((_TILE, D), lambda i: (i, 0)),
        out_shape=jax.ShapeDtypeStruct((T, D), jnp.float32),
    )(h, shared_gate.astype(jnp.bfloat16),
      shared_up.astype(jnp.bfloat16), shared_down.astype(jnp.bfloat16))

    dest_km = dest.reshape(T, _TOP_K).T.reshape(P)
    y_routed = _sc_row_gather(y_pad, dest_km, P).reshape(_TOP_K, T, D)
    w_km = topk_w.T
    out = shared_out + jnp.sum(w_km[:, :, None] * y_routed, axis=0)
    return out.reshape(orig_shape)
